# baseline (speedup 1.0000x reference)
import functools

import jax
import jax.numpy as jnp
from jax import lax
from jax.experimental import pallas as pl
from jax.experimental.pallas import tpu as pltpu
from jax.experimental.pallas import tpu_sc as plsc

B = 8
N = 20000
J = 17
P = 300
S = B * P
DJ = J * 3
NW = 32
RW = S // NW
RWP = 80
L = 16
G = 8
DJP = 56
NSUP = B * N // G


@functools.partial(
    pl.kernel,
    out_type=[
        jax.ShapeDtypeStruct((B,), jnp.int32),
        jax.ShapeDtypeStruct((NW, RWP * DJP), jnp.float32),
        jax.ShapeDtypeStruct((NW, RWP * 8), jnp.float32),
        jax.ShapeDtypeStruct((NW, RWP * 8), jnp.float32),
    ],
    mesh=plsc.VectorSubcoreMesh(core_axis_name="c", subcore_axis_name="s"),
    compiler_params=pltpu.CompilerParams(
        needs_layout_passes=False, use_tc_tiling_on_sc=False),
    scratch_types=[
        pltpu.VMEM((S * 3,), jnp.int32),
        pltpu.VMEM((RWP,), jnp.int32),
        pltpu.VMEM((RWP,), jnp.int32),
        pltpu.VMEM((RWP, DJ * G), jnp.float32),
        pltpu.VMEM((RWP, 4 * G), jnp.float32),
        pltpu.VMEM((RWP, G), jnp.float32),
        pltpu.VMEM((RWP * DJP,), jnp.float32),
        pltpu.VMEM((RWP * 8,), jnp.float32),
        pltpu.VMEM((RWP * 8,), jnp.float32),
        pltpu.VMEM((L,), jnp.int32),
        pltpu.VMEM((L,), jnp.int32),
        pltpu.SemaphoreType.DMA,
    ],
)
def _sc_route(sel_hbm, jt_hbm, bx_hbm, sc_hbm,
              num_out, j_out, b_out, s_out,
              sel_v, idx_v, sub_v, jsup, bsup, ssup,
              jbuf, bbuf, sbuf, lb_v, cnt_v, sem):
    cid = lax.axis_index("c")
    sid = lax.axis_index("s")
    wid = sid * 2 + cid

    pltpu.sync_copy(sel_hbm, sel_v)

    lane = lax.iota(jnp.int32, L)
    zeros_i = jnp.zeros((L,), jnp.int32)

    def bs_step(_, carry):
        lo, hi = carry
        unc = lo < hi
        mid = (lo + hi) // 2
        v = plsc.load_gather(sel_v, [jnp.minimum(mid, S - 1) * 3])
        lt = unc & (v < lane)
        return (jnp.where(lt, mid + 1, lo),
                jnp.where(unc & jnp.logical_not(lt), mid, hi))

    lo, _ = lax.fori_loop(
        0, 13, bs_step, (zeros_i, jnp.full((L,), S, jnp.int32)))
    lb_v[...] = lo
    nxt = plsc.load_gather(lb_v, [jnp.minimum(lane + 1, L - 1)])
    cnt_v[...] = nxt - lo

    @pl.when(wid == 0)
    def _():
        pltpu.sync_copy(cnt_v.at[pl.ds(0, B)], num_out)

    bw = wid // 4
    p0 = (wid % 4) * RW
    start_v = plsc.load_gather(lb_v, [jnp.full((L,), bw, jnp.int32)])
    cnt_bv = plsc.load_gather(cnt_v, [jnp.full((L,), bw, jnp.int32)])
    start_b = jnp.max(start_v)
    cnt_b = jnp.max(cnt_bv)
    nvalid = jnp.clip(cnt_b - p0, 0, RW)
    sel0 = start_b + p0

    for c in range(RWP // L):
        r = c * L + lane
        valid = r < nvalid
        seli = jnp.where(valid, sel0 + r, 0)
        xi = plsc.load_gather(sel_v, [seli * 3 + 2])
        src = jnp.where(valid, bw * N + xi, 0)
        idx_v[pl.ds(c * L, L)] = src // G
        sub_v[pl.ds(c * L, L)] = src - (src // G) * G

    cj = pltpu.async_copy(jt_hbm.at[idx_v], jsup, sem)
    cb = pltpu.async_copy(bx_hbm.at[idx_v], bsup, sem)
    cs = pltpu.async_copy(sc_hbm.at[idx_v], ssup, sem)
    cj.wait()
    cb.wait()
    cs.wait()

    zeros_f = jnp.zeros((L,), jnp.float32)

    def pack(sup, w, buf, wp, cols):
        def body(c, carry):
            f = c * L + lane
            r = f // wp
            cc = f - r * wp
            valid = (cc < cols) & (r < nvalid)
            sub = plsc.load_gather(sub_v, [r])
            col = sub * w + jnp.minimum(cc, cols - 1)
            v = plsc.load_gather(sup, [r, col])
            buf[pl.ds(c * L, L)] = jnp.where(valid, v, zeros_f)
            return carry

        lax.fori_loop(0, RWP * wp // L, body, 0)

    pack(jsup, DJ, jbuf, DJP, DJ)
    pack(bsup, 4, bbuf, 8, 4)
    pack(ssup, 1, sbuf, 8, 1)

    pltpu.sync_copy(jbuf, j_out.at[wid])
    pltpu.sync_copy(bbuf, b_out.at[wid])
    pltpu.sync_copy(sbuf, s_out.at[wid])


def kernel(pred_boxes, pred_scores, pred_joints, selected_indexes):
    jt = pred_joints.reshape(NSUP, DJ * G)
    bx = pred_boxes.reshape(NSUP, 4 * G)
    sc = pred_scores.reshape(NSUP, G)
    num, jr, br, sr = _sc_route(selected_indexes.reshape(S * 3), jt, bx, sc)
    num_predictions = num.reshape(B, 1)
    out_joints = jr.reshape(NW, RWP, DJP)[:, :RW, :DJ].reshape(B, P, J, 3)
    out_boxes = br.reshape(NW, RWP, 8)[:, :RW, :4].reshape(B, P, 4)
    out_scores = sr.reshape(NW, RWP, 8)[:, :RW, 0].reshape(B, P)
    return num_predictions, out_boxes, out_scores, out_joints


# speedup vs baseline: 72.4801x; 72.4801x over previous
import functools

import jax
import jax.numpy as jnp
from jax import lax
from jax.experimental import pallas as pl
from jax.experimental.pallas import tpu as pltpu
from jax.experimental.pallas import tpu_sc as plsc

B = 8
N = 20000
J = 17
P = 300
S = B * P
DJ = J * 3
NW = 32
RW = S // NW
L = 16
C = 128
JG = 3840
BG = 384
SG = 128


@functools.partial(
    pl.kernel,
    out_type=[
        jax.ShapeDtypeStruct((B,), jnp.int32),
        jax.ShapeDtypeStruct((NW, JG), jnp.float32),
        jax.ShapeDtypeStruct((NW, BG), jnp.float32),
        jax.ShapeDtypeStruct((NW, SG), jnp.float32),
    ],
    mesh=plsc.VectorSubcoreMesh(core_axis_name="c", subcore_axis_name="s"),
    compiler_params=pltpu.CompilerParams(
        needs_layout_passes=False, use_tc_tiling_on_sc=False),
    scratch_types=[
        pltpu.VMEM((S,), jnp.int32),
        pltpu.VMEM((S,), jnp.int32),
        pltpu.VMEM((RW + 5,), jnp.int32),
        pltpu.VMEM((JG,), jnp.int32),
        pltpu.VMEM((BG,), jnp.int32),
        pltpu.VMEM((SG,), jnp.int32),
        pltpu.VMEM((JG,), jnp.float32),
        pltpu.VMEM((BG,), jnp.float32),
        pltpu.VMEM((SG,), jnp.float32),
        pltpu.VMEM((L,), jnp.int32),
        pltpu.VMEM((L,), jnp.int32),
        pltpu.SemaphoreType.DMA,
    ],
)
def _sc_route(bi_hbm, xi_hbm, jt_hbm, bx_hbm, sc_hbm,
              num_out, j_out, b_out, s_out,
              bi_v, xi_v, xw_v, jidx, bidx, sidx,
              jbuf, bbuf, sbuf, lb_v, cnt_v, sem):
    cid = lax.axis_index("c")
    sid = lax.axis_index("s")
    wid = sid * 2 + cid

    cpb = pltpu.async_copy(bi_hbm, bi_v, sem)
    cpx = pltpu.async_copy(xi_hbm, xi_v, sem)
    cpb.wait()
    cpx.wait()

    lane = lax.iota(jnp.int32, L)
    zeros_i = jnp.zeros((L,), jnp.int32)

    def bs_step(_, carry):
        lo, hi = carry
        unc = lo < hi
        mid = (lo + hi) // 2
        v = plsc.load_gather(bi_v, [jnp.minimum(mid, S - 1)])
        lt = unc & (v < lane)
        return (jnp.where(lt, mid + 1, lo),
                jnp.where(unc & jnp.logical_not(lt), mid, hi))

    lo, _ = lax.fori_loop(
        0, 13, bs_step, (zeros_i, jnp.full((L,), S, jnp.int32)))
    lb_v[...] = lo
    nxt = plsc.load_gather(lb_v, [jnp.minimum(lane + 1, L - 1)])
    cnt_v[...] = nxt - lo

    @pl.when(wid == 0)
    def _():
        pltpu.sync_copy(cnt_v.at[pl.ds(0, B)], num_out)

    bw = wid // 4
    p0 = (wid % 4) * RW
    start_b = jnp.max(plsc.load_gather(lb_v, [jnp.full((L,), bw, jnp.int32)]))
    cnt_b = jnp.max(plsc.load_gather(cnt_v, [jnp.full((L,), bw, jnp.int32)]))
    nvalid = jnp.clip(cnt_b - p0, 0, RW)
    sel0 = start_b + p0

    for c in range(5):
        r = c * L + lane
        seli = jnp.minimum(sel0 + r, S - 1)
        xw_v[pl.ds(c * L, L)] = jnp.where(
            r < nvalid, plsc.load_gather(xi_v, [seli]), 0)

    def build(idx_ref, total, w, plane_stride, base):
        def body(ci, carry):
            f = ci * L + lane
            r = f // w
            k = f - r * w
            valid = (r < nvalid) & (f < RW * w)
            xr = plsc.load_gather(xw_v, [jnp.minimum(r, RW - 1)])
            idx = base + k * plane_stride + xr
            idx_ref[pl.ds(ci * L, L)] = jnp.where(valid, idx, 0)
            return carry

        lax.fori_loop(0, total // L, body, 0)

    build(jidx, JG, DJ, B * N, bw * N)
    build(bidx, BG, 4, N, bw * 4 * N)
    build(sidx, SG, 1, N, bw * N)

    copies = []
    for c in range(JG // C):
        copies.append(pltpu.async_copy(
            jt_hbm.at[jidx.at[pl.ds(c * C, C)]], jbuf.at[pl.ds(c * C, C)],
            sem))
    for c in range(BG // C):
        copies.append(pltpu.async_copy(
            bx_hbm.at[bidx.at[pl.ds(c * C, C)]], bbuf.at[pl.ds(c * C, C)],
            sem))
    copies.append(pltpu.async_copy(sc_hbm.at[sidx], sbuf, sem))
    for cp in copies:
        cp.wait()

    zeros_f = jnp.zeros((L,), jnp.float32)

    def zero_tail(buf, w, total):
        zf = nvalid * w

        def body(ci, carry):
            f = ci * L + lane
            v = buf[pl.ds(ci * L, L)]
            buf[pl.ds(ci * L, L)] = jnp.where(f >= zf, zeros_f, v)
            return carry

        lax.fori_loop(zf // L, total // L, body, 0)

    zero_tail(jbuf, DJ, JG)
    zero_tail(bbuf, 4, BG)
    zero_tail(sbuf, 1, SG)

    pltpu.sync_copy(jbuf, j_out.at[wid])
    pltpu.sync_copy(bbuf, b_out.at[wid])
    pltpu.sync_copy(sbuf, s_out.at[wid])


def kernel(pred_boxes, pred_scores, pred_joints, selected_indexes):
    jt = jnp.transpose(pred_joints, (2, 3, 0, 1)).reshape(-1)
    bx = jnp.transpose(pred_boxes, (0, 2, 1)).reshape(-1)
    sc = jnp.transpose(pred_scores, (0, 2, 1)).reshape(-1)
    bi = selected_indexes[:, 0]
    xi = selected_indexes[:, 2]
    num, jr, br, sr = _sc_route(bi, xi, jt, bx, sc)
    num_predictions = num.reshape(B, 1)
    out_joints = jr[:, :RW * DJ].reshape(S, DJ).reshape(B, P, J, 3)
    out_boxes = br[:, :RW * 4].reshape(S, 4).reshape(B, P, 4)
    out_scores = sr[:, :RW].reshape(B, P)
    return num_predictions, out_boxes, out_scores, out_joints


# speedup vs baseline: 117.1641x; 1.6165x over previous
import functools

import jax
import jax.numpy as jnp
from jax import lax
from jax.experimental import pallas as pl
from jax.experimental.pallas import tpu as pltpu
from jax.experimental.pallas import tpu_sc as plsc

B = 8
N = 20000
J = 17
P = 300
S = B * P
DJ = J * 3
NW = 32
L = 16
C = 128
PW = 384
PO = 304
MJ = 13


@functools.partial(
    pl.kernel,
    out_type=[
        jax.ShapeDtypeStruct((B,), jnp.int32),
        jax.ShapeDtypeStruct((DJ * B, PO), jnp.float32),
        jax.ShapeDtypeStruct((4 * B, PO), jnp.float32),
        jax.ShapeDtypeStruct((B, PO), jnp.float32),
    ],
    mesh=plsc.VectorSubcoreMesh(core_axis_name="c", subcore_axis_name="s"),
    compiler_params=pltpu.CompilerParams(
        needs_layout_passes=False, use_tc_tiling_on_sc=False),
    scratch_types=[
        pltpu.VMEM((S,), jnp.int32),
        pltpu.VMEM((S,), jnp.int32),
        pltpu.VMEM((PW,), jnp.int32),
        pltpu.VMEM(((MJ + 2) * PW,), jnp.float32),
        pltpu.VMEM((L,), jnp.int32),
        pltpu.VMEM((L,), jnp.int32),
        pltpu.SemaphoreType.DMA,
    ],
)
def _sc_route(bi_hbm, xi_hbm, jt_hbm, bx_hbm, sc_hbm,
              num_out, j_out, b_out, s_out,
              bi_v, xi_v, xw_v, data, lb_v, cnt_v, sem):
    cid = lax.axis_index("c")
    sid = lax.axis_index("s")
    wid = sid * 2 + cid

    cpb = pltpu.async_copy(bi_hbm, bi_v, sem)
    cpx = pltpu.async_copy(xi_hbm, xi_v, sem)
    cpb.wait()
    cpx.wait()

    lane = lax.iota(jnp.int32, L)
    zeros_i = jnp.zeros((L,), jnp.int32)

    def bs_step(_, carry):
        lo, hi = carry
        unc = lo < hi
        mid = (lo + hi) // 2
        v = plsc.load_gather(bi_v, [jnp.minimum(mid, S - 1)])
        lt = unc & (v < lane)
        return (jnp.where(lt, mid + 1, lo),
                jnp.where(unc & jnp.logical_not(lt), mid, hi))

    lo, _ = lax.fori_loop(
        0, 13, bs_step, (zeros_i, jnp.full((L,), S, jnp.int32)))
    lb_v[...] = lo
    nxt = plsc.load_gather(lb_v, [jnp.minimum(lane + 1, L - 1)])
    cnt_v[...] = nxt - lo

    @pl.when(wid == 0)
    def _():
        pltpu.sync_copy(cnt_v.at[pl.ds(0, B)], num_out)

    b = wid // 4
    q = wid % 4
    start_b = jnp.max(plsc.load_gather(lb_v, [jnp.full((L,), b, jnp.int32)]))
    cnt_b = jnp.max(plsc.load_gather(cnt_v, [jnp.full((L,), b, jnp.int32)]))
    nv = jnp.minimum(cnt_b, P)

    def xw_step(c, carry):
        p = c * L + lane
        seli = jnp.minimum(start_b + p, S - 1)
        xw_v[pl.ds(c * L, L)] = jnp.where(
            p < nv, plsc.load_gather(xi_v, [seli]), 0)
        return carry

    lax.fori_loop(0, PW // L, xw_step, 0)

    copies = []

    def plane_gathers(table, row, slot):
        for c in range(PW // C):
            copies.append(pltpu.async_copy(
                table.at[row].at[xw_v.at[pl.ds(c * C, C)]],
                data.at[pl.ds(slot * PW + c * C, C)], sem))

    for m in range(MJ):
        k = q + 4 * m
        plane_gathers(jt_hbm, jnp.minimum(k, DJ - 1) * B + b, m)

    plane_gathers(bx_hbm, b * 4 + q, MJ)
    plane_gathers(sc_hbm, b, MJ + 1)

    for cp in copies:
        cp.wait()

    zeros_f = jnp.zeros((L,), jnp.float32)

    def zero_tail(slot):
        base = slot * PW

        def body(c, carry):
            f = c * L + lane
            v = data[pl.ds(base + c * L, L)]
            data[pl.ds(base + c * L, L)] = jnp.where(f >= nv, zeros_f, v)
            return carry

        lax.fori_loop(nv // L, PW // L, body, 0)

    def plane_write(out, row, slot):
        zero_tail(slot)
        pltpu.sync_copy(data.at[pl.ds(slot * PW, PO)], out.at[row])

    for m in range(MJ):
        k = q + 4 * m

        @pl.when(k < DJ)
        def _(k=k, m=m):
            plane_write(j_out, k * B + b, m)

    plane_write(b_out, b * 4 + q, MJ)

    @pl.when(q == 0)
    def _():
        plane_write(s_out, b, MJ + 1)


def kernel(pred_boxes, pred_scores, pred_joints, selected_indexes):
    jt = jnp.transpose(pred_joints, (2, 3, 0, 1)).reshape(DJ * B, N)
    bx = jnp.transpose(pred_boxes, (0, 2, 1)).reshape(B * 4, N)
    sc = jnp.transpose(pred_scores, (0, 2, 1)).reshape(B, N)
    bi = selected_indexes[:, 0]
    xi = selected_indexes[:, 2]
    num, jr, br, sr = _sc_route(bi, xi, jt, bx, sc)
    num_predictions = num.reshape(B, 1)
    out_joints = jr[:, :P].reshape(J, 3, B, P).transpose(2, 3, 0, 1)
    out_boxes = br[:, :P].reshape(B, 4, P).transpose(0, 2, 1)
    out_scores = sr[:, :P]
    return num_predictions, out_boxes, out_scores, out_joints


# speedup vs baseline: 118.4253x; 1.0108x over previous
import functools

import jax
import jax.numpy as jnp
from jax import lax
from jax.experimental import pallas as pl
from jax.experimental.pallas import tpu as pltpu
from jax.experimental.pallas import tpu_sc as plsc

B = 8
N = 20000
J = 17
P = 300
S = B * P
DJ = J * 3
NW = 32
L = 16
C = 128
PW = 384
PO = 304
MJ = 13


@functools.partial(
    pl.kernel,
    out_type=[
        jax.ShapeDtypeStruct((B,), jnp.int32),
        jax.ShapeDtypeStruct((DJ * B, PO), jnp.float32),
        jax.ShapeDtypeStruct((4 * B, PO), jnp.float32),
        jax.ShapeDtypeStruct((B, PO), jnp.float32),
    ],
    mesh=plsc.VectorSubcoreMesh(core_axis_name="c", subcore_axis_name="s"),
    compiler_params=pltpu.CompilerParams(
        needs_layout_passes=False, use_tc_tiling_on_sc=False),
    scratch_types=[
        pltpu.VMEM((S,), jnp.int32),
        pltpu.VMEM((S,), jnp.int32),
        pltpu.VMEM((PW,), jnp.int32),
        pltpu.VMEM(((MJ + 2) * PW,), jnp.float32),
        pltpu.VMEM((L,), jnp.int32),
        pltpu.VMEM((L,), jnp.int32),
        pltpu.SemaphoreType.DMA,
        pltpu.SemaphoreType.DMA,
    ],
)
def _sc_route(bi_hbm, xi_hbm, jt_hbm, bx_hbm, sc_hbm,
              num_out, j_out, b_out, s_out,
              bi_v, xi_v, xw_v, data, lb_v, cnt_v, sem, wsem):
    cid = lax.axis_index("c")
    sid = lax.axis_index("s")
    wid = sid * 2 + cid

    cpb = pltpu.async_copy(bi_hbm, bi_v, sem)
    cpx = pltpu.async_copy(xi_hbm, xi_v, sem)
    cpb.wait()
    cpx.wait()

    lane = lax.iota(jnp.int32, L)
    zeros_i = jnp.zeros((L,), jnp.int32)

    def bs_step(_, carry):
        lo, hi = carry
        unc = lo < hi
        mid = (lo + hi) // 2
        v = plsc.load_gather(bi_v, [jnp.minimum(mid, S - 1)])
        lt = unc & (v < lane)
        return (jnp.where(lt, mid + 1, lo),
                jnp.where(unc & jnp.logical_not(lt), mid, hi))

    lo, _ = lax.fori_loop(
        0, 13, bs_step, (zeros_i, jnp.full((L,), S, jnp.int32)))
    lb_v[...] = lo
    nxt = plsc.load_gather(lb_v, [jnp.minimum(lane + 1, L - 1)])
    cnt_v[...] = nxt - lo

    @pl.when(wid == 0)
    def _():
        pltpu.sync_copy(cnt_v.at[pl.ds(0, B)], num_out)

    b = wid // 4
    q = wid % 4
    start_b = jnp.max(plsc.load_gather(lb_v, [jnp.full((L,), b, jnp.int32)]))
    cnt_b = jnp.max(plsc.load_gather(cnt_v, [jnp.full((L,), b, jnp.int32)]))
    nv = jnp.minimum(cnt_b, P)

    def xw_step(c, carry):
        p = c * L + lane
        seli = jnp.minimum(start_b + p, S - 1)
        xw_v[pl.ds(c * L, L)] = jnp.where(
            p < nv, plsc.load_gather(xi_v, [seli]), 0)
        return carry

    lax.fori_loop(0, PW // L, xw_step, 0)

    copies = []

    def plane_gathers(table, row, slot):
        for c in range(PW // C):
            copies.append(pltpu.async_copy(
                table.at[row].at[xw_v.at[pl.ds(c * C, C)]],
                data.at[pl.ds(slot * PW + c * C, C)], sem))

    for m in range(MJ):
        k = q + 4 * m
        plane_gathers(jt_hbm, jnp.minimum(k, DJ - 1) * B + b, m)

    plane_gathers(bx_hbm, b * 4 + q, MJ)
    plane_gathers(sc_hbm, b, MJ + 1)

    for cp in copies:
        cp.wait()

    zeros_f = jnp.zeros((L,), jnp.float32)

    def zero_tail(slot):
        base = slot * PW

        def body(c, carry):
            f = c * L + lane
            v = data[pl.ds(base + c * L, L)]
            data[pl.ds(base + c * L, L)] = jnp.where(f >= nv, zeros_f, v)
            return carry

        lax.fori_loop(nv // L, PW // L, body, 0)

    wcopies = []

    def plane_write(out, row, slot):
        zero_tail(slot)
        wcopies.append(pltpu.async_copy(
            data.at[pl.ds(slot * PW, PO)], out.at[row], wsem))

    def plane_write_sync(out, row, slot):
        zero_tail(slot)
        pltpu.sync_copy(data.at[pl.ds(slot * PW, PO)], out.at[row])

    for m in range(MJ - 1):
        k = q + 4 * m
        plane_write(j_out, k * B + b, m)

    k12 = q + 4 * (MJ - 1)

    @pl.when(k12 < DJ)
    def _():
        plane_write_sync(j_out, k12 * B + b, MJ - 1)

    plane_write(b_out, b * 4 + q, MJ)

    @pl.when(q == 0)
    def _():
        plane_write_sync(s_out, b, MJ + 1)

    for wc in wcopies:
        wc.wait()


def kernel(pred_boxes, pred_scores, pred_joints, selected_indexes):
    jt = jnp.transpose(pred_joints, (2, 3, 0, 1)).reshape(DJ * B, N)
    bx = jnp.transpose(pred_boxes, (0, 2, 1)).reshape(B * 4, N)
    sc = jnp.transpose(pred_scores, (0, 2, 1)).reshape(B, N)
    bi = selected_indexes[:, 0]
    xi = selected_indexes[:, 2]
    num, jr, br, sr = _sc_route(bi, xi, jt, bx, sc)
    num_predictions = num.reshape(B, 1)
    out_joints = jr[:, :P].reshape(J, 3, B, P).transpose(2, 3, 0, 1)
    out_boxes = br[:, :P].reshape(B, 4, P).transpose(0, 2, 1)
    out_scores = sr[:, :P]
    return num_predictions, out_boxes, out_scores, out_joints


# speedup vs baseline: 125.2198x; 1.0574x over previous
import functools

import jax
import jax.numpy as jnp
from jax import lax
from jax.experimental import pallas as pl
from jax.experimental.pallas import tpu as pltpu
from jax.experimental.pallas import tpu_sc as plsc

B = 8
N = 20000
J = 17
P = 300
S = B * P
DJ = J * 3
NW = 32
L = 16
C = 128
PW = 384
PO = 304
MJ = 13


@functools.partial(
    pl.kernel,
    out_type=[
        jax.ShapeDtypeStruct((B,), jnp.int32),
        jax.ShapeDtypeStruct((DJ * B, PO), jnp.float32),
        jax.ShapeDtypeStruct((4 * B, PO), jnp.float32),
        jax.ShapeDtypeStruct((B, PO), jnp.float32),
    ],
    mesh=plsc.VectorSubcoreMesh(core_axis_name="c", subcore_axis_name="s"),
    compiler_params=pltpu.CompilerParams(
        needs_layout_passes=False, use_tc_tiling_on_sc=False),
    scratch_types=[
        pltpu.VMEM((S,), jnp.int32),
        pltpu.VMEM((S,), jnp.int32),
        pltpu.VMEM((PW,), jnp.int32),
        pltpu.VMEM(((MJ + 2) * PW,), jnp.float32),
        pltpu.VMEM((L,), jnp.int32),
        pltpu.VMEM((L,), jnp.int32),
        pltpu.SemaphoreType.DMA,
        pltpu.SemaphoreType.DMA,
    ],
)
def _sc_route(bi_hbm, xi_hbm, jt_hbm, bx_hbm, sc_hbm,
              num_out, j_out, b_out, s_out,
              bi_v, xi_v, xw_v, data, lb_v, cnt_v, sem, wsem):
    cid = lax.axis_index("c")
    sid = lax.axis_index("s")
    wid = sid * 2 + cid

    cpb = pltpu.async_copy(bi_hbm, bi_v, sem)
    cpx = pltpu.async_copy(xi_hbm, xi_v, sem)
    cpb.wait()
    cpx.wait()

    lane = lax.iota(jnp.int32, L)
    zeros_i = jnp.zeros((L,), jnp.int32)

    def bs_step(_, carry):
        lo, hi = carry
        unc = lo < hi
        mid = (lo + hi) // 2
        v = plsc.load_gather(bi_v, [jnp.minimum(mid, S - 1)])
        lt = unc & (v < lane)
        return (jnp.where(lt, mid + 1, lo),
                jnp.where(unc & jnp.logical_not(lt), mid, hi))

    lo, _ = lax.fori_loop(
        0, 13, bs_step, (zeros_i, jnp.full((L,), S, jnp.int32)))
    lb_v[...] = lo
    nxt = plsc.load_gather(lb_v, [jnp.minimum(lane + 1, L - 1)])
    cnt_v[...] = nxt - lo

    @pl.when(wid == 0)
    def _():
        pltpu.sync_copy(cnt_v.at[pl.ds(0, B)], num_out)

    b = wid // 4
    q = wid % 4
    start_b = jnp.max(plsc.load_gather(lb_v, [jnp.full((L,), b, jnp.int32)]))
    cnt_b = jnp.max(plsc.load_gather(cnt_v, [jnp.full((L,), b, jnp.int32)]))
    nv = jnp.minimum(cnt_b, P)

    def xw_step(c, carry):
        p = c * L + lane
        seli = jnp.minimum(start_b + p, S - 1)
        xw_v[pl.ds(c * L, L)] = jnp.where(
            p < nv, plsc.load_gather(xi_v, [seli]), 0)
        return carry

    lax.fori_loop(0, PW // L, xw_step, 0)

    copies = []

    def plane_gathers(table, row, slot):
        for c, w in ((0, C), (1, C), (2, P - 2 * C)):
            copies.append(pltpu.async_copy(
                table.at[row].at[xw_v.at[pl.ds(c * C, w)]],
                data.at[pl.ds(slot * PW + c * C, w)], sem))

    for m in range(MJ):
        k = q + 4 * m
        plane_gathers(jt_hbm, jnp.minimum(k, DJ - 1) * B + b, m)

    plane_gathers(bx_hbm, b * 4 + q, MJ)
    plane_gathers(sc_hbm, b, MJ + 1)

    for cp in copies:
        cp.wait()

    zeros_f = jnp.zeros((L,), jnp.float32)

    def zero_tail(slot):
        base = slot * PW

        def body(c, carry):
            f = c * L + lane
            v = data[pl.ds(base + c * L, L)]
            data[pl.ds(base + c * L, L)] = jnp.where(f >= nv, zeros_f, v)
            return carry

        lax.fori_loop(nv // L, PW // L, body, 0)

    wcopies = []

    def plane_write(out, row, slot):
        zero_tail(slot)
        wcopies.append(pltpu.async_copy(
            data.at[pl.ds(slot * PW, PO)], out.at[row], wsem))

    def plane_write_sync(out, row, slot):
        zero_tail(slot)
        pltpu.sync_copy(data.at[pl.ds(slot * PW, PO)], out.at[row])

    for m in range(MJ - 1):
        k = q + 4 * m
        plane_write(j_out, k * B + b, m)

    k12 = q + 4 * (MJ - 1)

    @pl.when(k12 < DJ)
    def _():
        plane_write_sync(j_out, k12 * B + b, MJ - 1)

    plane_write(b_out, b * 4 + q, MJ)

    @pl.when(q == 0)
    def _():
        plane_write_sync(s_out, b, MJ + 1)

    for wc in wcopies:
        wc.wait()


def kernel(pred_boxes, pred_scores, pred_joints, selected_indexes):
    jt = jnp.transpose(pred_joints, (2, 3, 0, 1)).reshape(DJ * B, N)
    bx = jnp.transpose(pred_boxes, (0, 2, 1)).reshape(B * 4, N)
    sc = jnp.transpose(pred_scores, (0, 2, 1)).reshape(B, N)
    bi = selected_indexes[:, 0]
    xi = selected_indexes[:, 2]
    num, jr, br, sr = _sc_route(bi, xi, jt, bx, sc)
    num_predictions = num.reshape(B, 1)
    out_joints = jr[:, :P].reshape(J, 3, B, P).transpose(2, 3, 0, 1)
    out_boxes = br[:, :P].reshape(B, 4, P).transpose(0, 2, 1)
    out_scores = sr[:, :P]
    return num_predictions, out_boxes, out_scores, out_joints
